# Initial kernel scaffold; baseline (speedup 1.0000x reference)
import functools

import jax
import jax.numpy as jnp
from jax import lax
from jax.experimental import pallas as pl
from jax.experimental.pallas import tpu as pltpu
from jax.experimental.pallas import tpu_sc as plsc

B, L = 4096, 50
EMB_DIM, HIDDEN = 128, 512
CHUNK = 128
NCHUNK = (B // 32) * L // CHUNK
ROWS_PER_W = B // 32


def _sc_pool(idx_t, idx_k, idx_v, type_table, key_table, val_table,
             dst_map, zeros):
  mesh = plsc.VectorSubcoreMesh(core_axis_name="c", subcore_axis_name="s")
  out_t = [jax.ShapeDtypeStruct((B, EMB_DIM), jnp.float32)] * 3

  @functools.partial(
      pl.kernel,
      mesh=mesh,
      out_type=out_t,
      scratch_types=[
          pltpu.VMEM((NCHUNK, CHUNK), jnp.int32),
          pltpu.VMEM((NCHUNK, CHUNK), jnp.int32),
          pltpu.VMEM((CHUNK, EMB_DIM), jnp.float32),
          pltpu.VMEM((ROWS_PER_W, EMB_DIM), jnp.float32),
          pltpu.SemaphoreType.DMA,
          pltpu.SemaphoreType.DMA,
      ],
  )
  def k(idx_t_hbm, idx_k_hbm, idx_v_hbm, ttab, ktab, vtab, dmap_hbm,
        zeros_hbm, o_t, o_k, o_v, idx_vm, dmap_vm, rows_vm, acc_vm,
        gsem, ssem):
    wid = lax.axis_index("c") * 16 + lax.axis_index("s")
    pltpu.sync_copy(dmap_hbm, dmap_vm)

    for idx_hbm, tab, out in ((idx_t_hbm, ttab, o_t),
                              (idx_k_hbm, ktab, o_k),
                              (idx_v_hbm, vtab, o_v)):
      pltpu.sync_copy(idx_hbm.at[pl.ds(wid * NCHUNK, NCHUNK)], idx_vm)
      pltpu.sync_copy(zeros_hbm, acc_vm)

      def body(c, carry):
        pltpu.async_copy(tab.at[idx_vm.at[c]], rows_vm, gsem).wait()
        pltpu.async_copy(rows_vm, acc_vm.at[dmap_vm.at[c]], ssem,
                         add=True).wait()
        return carry

      lax.fori_loop(0, NCHUNK, body, 0)
      pltpu.sync_copy(acc_vm, out.at[pl.ds(wid * ROWS_PER_W, ROWS_PER_W)])

  return k(idx_t, idx_k, idx_v, type_table, key_table, val_table,
           dst_map, zeros)


def _tc_head(sum_t, sum_k, sum_v, wt_t, wk_t, wv_t, b2):
  BM = 512
  grid = (B // BM,)

  def body(st_r, sk_r, sv_r, wt_r, wk_r, wv_r, b_r, o_r):
    acc = jnp.dot(st_r[...], wt_r[...], preferred_element_type=jnp.float32)
    acc += jnp.dot(sk_r[...], wk_r[...], preferred_element_type=jnp.float32)
    acc += jnp.dot(sv_r[...], wv_r[...], preferred_element_type=jnp.float32)
    o_r[...] = jnp.maximum(acc * (1.0 / L) + b_r[...], 0.0)

  return pl.pallas_call(
      body,
      grid=grid,
      in_specs=[
          pl.BlockSpec((BM, EMB_DIM), lambda i: (i, 0)),
          pl.BlockSpec((BM, EMB_DIM), lambda i: (i, 0)),
          pl.BlockSpec((BM, EMB_DIM), lambda i: (i, 0)),
          pl.BlockSpec((EMB_DIM, HIDDEN), lambda i: (0, 0)),
          pl.BlockSpec((EMB_DIM, HIDDEN), lambda i: (0, 0)),
          pl.BlockSpec((EMB_DIM, HIDDEN), lambda i: (0, 0)),
          pl.BlockSpec((1, HIDDEN), lambda i: (0, 0)),
      ],
      out_specs=pl.BlockSpec((BM, HIDDEN), lambda i: (i, 0)),
      out_shape=jax.ShapeDtypeStruct((B, HIDDEN), jnp.float32),
  )(sum_t, sum_k, sum_v, wt_t, wk_t, wv_t, b2)


def kernel(sol_type_idx, sol_key_idx, sol_val_idx, type_table, key_table,
           val_table, W, b):
  idx_t = sol_type_idx.astype(jnp.int32).reshape(-1, CHUNK)
  idx_k = sol_key_idx.astype(jnp.int32).reshape(-1, CHUNK)
  idx_v = sol_val_idx.astype(jnp.int32).reshape(-1, CHUNK)
  dst_map = (jnp.arange(ROWS_PER_W * L, dtype=jnp.int32) // L).reshape(
      NCHUNK, CHUNK)
  zeros = jnp.zeros((ROWS_PER_W, EMB_DIM), jnp.float32)

  sum_t, sum_k, sum_v = _sc_pool(idx_t, idx_k, idx_v, type_table, key_table,
                                 val_table, dst_map, zeros)

  wT = W.T
  wt_t = wT[:EMB_DIM]
  wk_t = wT[EMB_DIM:2 * EMB_DIM]
  wv_t = wT[2 * EMB_DIM:]
  return _tc_head(sum_t, sum_k, sum_v, wt_t, wk_t, wv_t, b.reshape(1, HIDDEN))


# speedup vs baseline: 8.2678x; 8.2678x over previous
import functools

import jax
import jax.numpy as jnp
from jax import lax
from jax.experimental import pallas as pl
from jax.experimental.pallas import tpu as pltpu
from jax.experimental.pallas import tpu_sc as plsc

B, L = 4096, 50
EMB_DIM, HIDDEN = 128, 512
CHUNK = 128
NCHUNK = (B // 32) * L // CHUNK
ROWS_PER_W = B // 32


def _sc_pool(idx_t, idx_k, idx_v, type_table, key_table, val_table,
             dst_map, zeros):
  mesh = plsc.VectorSubcoreMesh(core_axis_name="c", subcore_axis_name="s")
  out_t = [jax.ShapeDtypeStruct((B, EMB_DIM), jnp.float32)] * 3

  @functools.partial(
      pl.kernel,
      mesh=mesh,
      out_type=out_t,
      scratch_types=[
          pltpu.VMEM((NCHUNK, CHUNK), jnp.int32),
          pltpu.VMEM((NCHUNK, CHUNK), jnp.int32),
          pltpu.VMEM((CHUNK, EMB_DIM), jnp.float32),
          pltpu.VMEM_SHARED((16 * ROWS_PER_W, EMB_DIM), jnp.float32),
          pltpu.SemaphoreType.DMA,
          pltpu.SemaphoreType.DMA,
      ],
  )
  def k(idx_t_hbm, idx_k_hbm, idx_v_hbm, ttab, ktab, vtab, dmap_hbm,
        zeros_hbm, o_t, o_k, o_v, idx_vm, dmap_vm, rows_vm, acc_sh,
        gsem, ssem):
    s = lax.axis_index("s")
    wid = lax.axis_index("c") * 16 + s
    pltpu.sync_copy(dmap_hbm.at[s], dmap_vm)

    for idx_hbm, tab, out in ((idx_t_hbm, ttab, o_t),
                              (idx_k_hbm, ktab, o_k),
                              (idx_v_hbm, vtab, o_v)):
      pltpu.sync_copy(idx_hbm.at[wid], idx_vm)
      pltpu.sync_copy(zeros_hbm, acc_sh.at[pl.ds(s * ROWS_PER_W, ROWS_PER_W)])

      def body(c, carry):
        pltpu.async_copy(tab.at[idx_vm.at[c]], rows_vm, gsem).wait()
        pltpu.async_copy(rows_vm, acc_sh.at[dmap_vm.at[c]], ssem,
                         add=True).wait()
        return carry

      lax.fori_loop(0, NCHUNK, body, 0)
      pltpu.sync_copy(acc_sh.at[pl.ds(s * ROWS_PER_W, ROWS_PER_W)],
                      out.at[pl.ds(wid * ROWS_PER_W, ROWS_PER_W)])

  return k(idx_t, idx_k, idx_v, type_table, key_table, val_table,
           dst_map, zeros)


def _tc_head(sum_t, sum_k, sum_v, wt_t, wk_t, wv_t, b2):
  BM = 512
  grid = (B // BM,)

  def body(st_r, sk_r, sv_r, wt_r, wk_r, wv_r, b_r, o_r):
    acc = jnp.dot(st_r[...], wt_r[...], preferred_element_type=jnp.float32)
    acc += jnp.dot(sk_r[...], wk_r[...], preferred_element_type=jnp.float32)
    acc += jnp.dot(sv_r[...], wv_r[...], preferred_element_type=jnp.float32)
    o_r[...] = jnp.maximum(acc * (1.0 / L) + b_r[...], 0.0)

  return pl.pallas_call(
      body,
      grid=grid,
      in_specs=[
          pl.BlockSpec((BM, EMB_DIM), lambda i: (i, 0)),
          pl.BlockSpec((BM, EMB_DIM), lambda i: (i, 0)),
          pl.BlockSpec((BM, EMB_DIM), lambda i: (i, 0)),
          pl.BlockSpec((EMB_DIM, HIDDEN), lambda i: (0, 0)),
          pl.BlockSpec((EMB_DIM, HIDDEN), lambda i: (0, 0)),
          pl.BlockSpec((EMB_DIM, HIDDEN), lambda i: (0, 0)),
          pl.BlockSpec((1, HIDDEN), lambda i: (0, 0)),
      ],
      out_specs=pl.BlockSpec((BM, HIDDEN), lambda i: (i, 0)),
      out_shape=jax.ShapeDtypeStruct((B, HIDDEN), jnp.float32),
  )(sum_t, sum_k, sum_v, wt_t, wk_t, wv_t, b2)


def kernel(sol_type_idx, sol_key_idx, sol_val_idx, type_table, key_table,
           val_table, W, b):
  idx_t = sol_type_idx.astype(jnp.int32).reshape(32, NCHUNK, CHUNK)
  idx_k = sol_key_idx.astype(jnp.int32).reshape(32, NCHUNK, CHUNK)
  idx_v = sol_val_idx.astype(jnp.int32).reshape(32, NCHUNK, CHUNK)
  pos = jnp.arange(ROWS_PER_W * L, dtype=jnp.int32) // L
  dst_map = (jnp.arange(16, dtype=jnp.int32)[:, None] * ROWS_PER_W +
             pos[None, :]).reshape(16, NCHUNK, CHUNK)
  zeros = jnp.zeros((ROWS_PER_W, EMB_DIM), jnp.float32)

  sum_t, sum_k, sum_v = _sc_pool(idx_t, idx_k, idx_v, type_table, key_table,
                                 val_table, dst_map, zeros)

  wT = W.T
  wt_t = wT[:EMB_DIM]
  wk_t = wT[EMB_DIM:2 * EMB_DIM]
  wv_t = wT[2 * EMB_DIM:]
  return _tc_head(sum_t, sum_k, sum_v, wt_t, wk_t, wv_t, b.reshape(1, HIDDEN))


# speedup vs baseline: 10.5688x; 1.2783x over previous
import functools

import jax
import jax.numpy as jnp
from jax import lax
from jax.experimental import pallas as pl
from jax.experimental.pallas import tpu as pltpu
from jax.experimental.pallas import tpu_sc as plsc

B, L = 4096, 50
EMB_DIM, HIDDEN = 128, 512
CHUNK = 128
NCHUNK = (B // 32) * L // CHUNK
ROWS_PER_W = B // 32


def _sc_pool(idx_t, idx_k, idx_v, type_table, key_table, val_table,
             dst_map, zeros):
  mesh = plsc.VectorSubcoreMesh(core_axis_name="c", subcore_axis_name="s")
  out_t = [jax.ShapeDtypeStruct((B, EMB_DIM), jnp.float32)] * 3

  @functools.partial(
      pl.kernel,
      mesh=mesh,
      out_type=out_t,
      scratch_types=[
          pltpu.VMEM((NCHUNK, CHUNK), jnp.int32),
          pltpu.VMEM((NCHUNK, CHUNK), jnp.int32),
          pltpu.VMEM((2 * CHUNK, EMB_DIM), jnp.float32),
          pltpu.VMEM_SHARED((16 * ROWS_PER_W, EMB_DIM), jnp.float32),
          pltpu.SemaphoreType.DMA,
          pltpu.SemaphoreType.DMA,
      ],
  )
  def k(idx_t_hbm, idx_k_hbm, idx_v_hbm, ttab, ktab, vtab, dmap_hbm,
        zeros_hbm, o_t, o_k, o_v, idx_vm, dmap_vm, rows_vm, acc_sh,
        gsem, ssem):
    s = lax.axis_index("s")
    wid = lax.axis_index("c") * 16 + s
    pltpu.sync_copy(dmap_hbm.at[s], dmap_vm)

    for idx_hbm, tab, out in ((idx_t_hbm, ttab, o_t),
                              (idx_k_hbm, ktab, o_k),
                              (idx_v_hbm, vtab, o_v)):
      pltpu.sync_copy(idx_hbm.at[wid], idx_vm)
      pltpu.sync_copy(zeros_hbm, acc_sh.at[pl.ds(s * ROWS_PER_W, ROWS_PER_W)])

      pltpu.async_copy(tab.at[idx_vm.at[0]],
                       rows_vm.at[pl.ds(0, CHUNK)], gsem)

      def body(c, carry):
        off = lax.rem(c, 2) * CHUNK
        buf = rows_vm.at[pl.ds(off, CHUNK)]
        pltpu.make_async_copy(tab.at[idx_vm.at[c]], buf, gsem).wait()

        @pl.when(c >= 1)
        def _():
          pltpu.make_async_copy(buf, acc_sh.at[dmap_vm.at[c]], ssem).wait()

        @pl.when(c <= NCHUNK - 2)
        def _():
          nbuf = rows_vm.at[pl.ds((CHUNK - off), CHUNK)]
          pltpu.async_copy(tab.at[idx_vm.at[c + 1]], nbuf, gsem)

        pltpu.async_copy(buf, acc_sh.at[dmap_vm.at[c]], ssem, add=True)
        return carry

      lax.fori_loop(0, NCHUNK, body, 0)
      pltpu.make_async_copy(rows_vm.at[pl.ds(0, CHUNK)],
                            acc_sh.at[dmap_vm.at[0]], ssem).wait()
      pltpu.sync_copy(acc_sh.at[pl.ds(s * ROWS_PER_W, ROWS_PER_W)],
                      out.at[pl.ds(wid * ROWS_PER_W, ROWS_PER_W)])

  return k(idx_t, idx_k, idx_v, type_table, key_table, val_table,
           dst_map, zeros)


def _tc_head(sum_t, sum_k, sum_v, wt_t, wk_t, wv_t, b2):
  BM = 512
  grid = (B // BM,)

  def body(st_r, sk_r, sv_r, wt_r, wk_r, wv_r, b_r, o_r):
    acc = jnp.dot(st_r[...], wt_r[...], preferred_element_type=jnp.float32)
    acc += jnp.dot(sk_r[...], wk_r[...], preferred_element_type=jnp.float32)
    acc += jnp.dot(sv_r[...], wv_r[...], preferred_element_type=jnp.float32)
    o_r[...] = jnp.maximum(acc * (1.0 / L) + b_r[...], 0.0)

  return pl.pallas_call(
      body,
      grid=grid,
      in_specs=[
          pl.BlockSpec((BM, EMB_DIM), lambda i: (i, 0)),
          pl.BlockSpec((BM, EMB_DIM), lambda i: (i, 0)),
          pl.BlockSpec((BM, EMB_DIM), lambda i: (i, 0)),
          pl.BlockSpec((EMB_DIM, HIDDEN), lambda i: (0, 0)),
          pl.BlockSpec((EMB_DIM, HIDDEN), lambda i: (0, 0)),
          pl.BlockSpec((EMB_DIM, HIDDEN), lambda i: (0, 0)),
          pl.BlockSpec((1, HIDDEN), lambda i: (0, 0)),
      ],
      out_specs=pl.BlockSpec((BM, HIDDEN), lambda i: (i, 0)),
      out_shape=jax.ShapeDtypeStruct((B, HIDDEN), jnp.float32),
  )(sum_t, sum_k, sum_v, wt_t, wk_t, wv_t, b2)


def kernel(sol_type_idx, sol_key_idx, sol_val_idx, type_table, key_table,
           val_table, W, b):
  idx_t = sol_type_idx.astype(jnp.int32).reshape(32, NCHUNK, CHUNK)
  idx_k = sol_key_idx.astype(jnp.int32).reshape(32, NCHUNK, CHUNK)
  idx_v = sol_val_idx.astype(jnp.int32).reshape(32, NCHUNK, CHUNK)
  pos = jnp.arange(ROWS_PER_W * L, dtype=jnp.int32) // L
  dst_map = (jnp.arange(16, dtype=jnp.int32)[:, None] * ROWS_PER_W +
             pos[None, :]).reshape(16, NCHUNK, CHUNK)
  zeros = jnp.zeros((ROWS_PER_W, EMB_DIM), jnp.float32)

  sum_t, sum_k, sum_v = _sc_pool(idx_t, idx_k, idx_v, type_table, key_table,
                                 val_table, dst_map, zeros)

  wT = W.T
  wt_t = wT[:EMB_DIM]
  wk_t = wT[EMB_DIM:2 * EMB_DIM]
  wv_t = wT[2 * EMB_DIM:]
  return _tc_head(sum_t, sum_k, sum_v, wt_t, wk_t, wv_t, b.reshape(1, HIDDEN))


# speedup vs baseline: 10.9934x; 1.0402x over previous
import functools

import jax
import jax.numpy as jnp
from jax import lax
from jax.experimental import pallas as pl
from jax.experimental.pallas import tpu as pltpu
from jax.experimental.pallas import tpu_sc as plsc

B, L = 4096, 50
EMB_DIM, HIDDEN = 128, 512
VPAD = 1024
CHUNK = 100
NCHUNK = (B // 32) * L // CHUNK
ROWS_PER_W = B // 32
NBUF = 2
PASS_ROWS = 32
NPASS = ROWS_PER_W // PASS_ROWS
PASS_POS = PASS_ROWS * L
PASS_PAD = 1664


def _sc_pool(idx_v, dstc, val_table, dmap, zeros2d, zeros1d):
  mesh = plsc.VectorSubcoreMesh(core_axis_name="c", subcore_axis_name="s")
  out_t = [
      jax.ShapeDtypeStruct((B * VPAD,), jnp.float32),
      jax.ShapeDtypeStruct((B * VPAD,), jnp.float32),
      jax.ShapeDtypeStruct((B, EMB_DIM), jnp.float32),
  ]

  @functools.partial(
      pl.kernel,
      mesh=mesh,
      out_type=out_t,
      compiler_params=pltpu.CompilerParams(needs_layout_passes=False),
      scratch_types=[
          pltpu.VMEM((NCHUNK, CHUNK), jnp.int32),
          pltpu.VMEM((NCHUNK, CHUNK), jnp.int32),
          pltpu.VMEM((PASS_PAD // 128, 8, 16), jnp.int32),
          pltpu.VMEM((PASS_ROWS * VPAD,), jnp.float32),
          pltpu.VMEM((NBUF * CHUNK, EMB_DIM), jnp.float32),
          pltpu.VMEM_SHARED((16 * ROWS_PER_W, EMB_DIM), jnp.float32),
      ] + [pltpu.SemaphoreType.DMA] * (2 * NBUF),
  )
  def k(idx_v_hbm, dstc_hbm, vtab, dmap_hbm, zeros2d_hbm, zeros1d_hbm,
        o_ct, o_ck, o_sv, idx_vm, dmap_vm, dst_vm, cnt_vm, rows_vm, acc_sh,
        *sems):
    gsems = sems[:NBUF]
    ssems = sems[NBUF:]
    s = lax.axis_index("s")
    wid = lax.axis_index("c") * 16 + s
    ones16 = jnp.ones((16,), jnp.float32)
    bufs = [rows_vm.at[pl.ds(j * CHUNK, CHUNK)] for j in range(NBUF)]

    for t, cout in ((0, o_ct), (1, o_ck)):
      for q in range(NPASS):
        pltpu.sync_copy(zeros1d_hbm, cnt_vm)
        pltpu.sync_copy(dstc_hbm.at[t].at[wid].at[q], dst_vm)

        def cbody(r, carry):
          for j in range(8):
            idx16 = dst_vm[r, j]
            plsc.addupdate_scatter(cnt_vm, [idx16], ones16)
          return carry

        lax.fori_loop(0, 12, cbody, 0)
        for j in range(4):
          idx16 = dst_vm[12, j]
          plsc.addupdate_scatter(cnt_vm, [idx16], ones16)

        pltpu.sync_copy(
            cnt_vm,
            cout.at[pl.ds((wid * ROWS_PER_W + q * PASS_ROWS) * VPAD,
                          PASS_ROWS * VPAD)])

    pltpu.sync_copy(dmap_hbm.at[s], dmap_vm)
    pltpu.sync_copy(idx_v_hbm.at[wid], idx_vm)
    pltpu.sync_copy(zeros2d_hbm, acc_sh.at[pl.ds(s * ROWS_PER_W, ROWS_PER_W)])

    gsem, ssem = gsems[0], ssems[0]
    pltpu.async_copy(vtab.at[idx_vm.at[0]], bufs[0], gsem)

    def body(c, carry):
      off = lax.rem(c, 2) * CHUNK
      buf = rows_vm.at[pl.ds(off, CHUNK)]
      pltpu.make_async_copy(vtab.at[idx_vm.at[c]], buf, gsem).wait()

      @pl.when(c >= 1)
      def _():
        pltpu.make_async_copy(buf, acc_sh.at[dmap_vm.at[c]], ssem).wait()

      @pl.when(c <= NCHUNK - 2)
      def _():
        nbuf = rows_vm.at[pl.ds((CHUNK - off), CHUNK)]
        pltpu.async_copy(vtab.at[idx_vm.at[c + 1]], nbuf, gsem)

      pltpu.async_copy(buf, acc_sh.at[dmap_vm.at[c]], ssem, add=True)
      return carry

    lax.fori_loop(0, NCHUNK, body, 0)
    pltpu.make_async_copy(bufs[0], acc_sh.at[dmap_vm.at[0]], ssem).wait()
    pltpu.sync_copy(acc_sh.at[pl.ds(s * ROWS_PER_W, ROWS_PER_W)],
                    o_sv.at[pl.ds(wid * ROWS_PER_W, ROWS_PER_W)])

  return k(idx_v, dstc, val_table, dmap, zeros2d, zeros1d)


def _tc_head(ct, ck, sv, tp, kp, wt_t, wk_t, wv_t, b2):
  BM = 512
  grid = (B // BM,)

  def body(ct_r, ck_r, sv_r, tp_r, kp_r, wt_r, wk_r, wv_r, b_r, o_r):
    st = jnp.dot(ct_r[...], tp_r[...], preferred_element_type=jnp.float32)
    sk = jnp.dot(ck_r[...], kp_r[...], preferred_element_type=jnp.float32)
    acc = jnp.dot(st, wt_r[...], preferred_element_type=jnp.float32)
    acc += jnp.dot(sk, wk_r[...], preferred_element_type=jnp.float32)
    acc += jnp.dot(sv_r[...], wv_r[...], preferred_element_type=jnp.float32)
    o_r[...] = jnp.maximum(acc * (1.0 / L) + b_r[...], 0.0)

  return pl.pallas_call(
      body,
      grid=grid,
      in_specs=[
          pl.BlockSpec((BM, VPAD), lambda i: (i, 0)),
          pl.BlockSpec((BM, VPAD), lambda i: (i, 0)),
          pl.BlockSpec((BM, EMB_DIM), lambda i: (i, 0)),
          pl.BlockSpec((VPAD, EMB_DIM), lambda i: (0, 0)),
          pl.BlockSpec((VPAD, EMB_DIM), lambda i: (0, 0)),
          pl.BlockSpec((EMB_DIM, HIDDEN), lambda i: (0, 0)),
          pl.BlockSpec((EMB_DIM, HIDDEN), lambda i: (0, 0)),
          pl.BlockSpec((EMB_DIM, HIDDEN), lambda i: (0, 0)),
          pl.BlockSpec((1, HIDDEN), lambda i: (0, 0)),
      ],
      out_specs=pl.BlockSpec((BM, HIDDEN), lambda i: (i, 0)),
      out_shape=jax.ShapeDtypeStruct((B, HIDDEN), jnp.float32),
  )(ct, ck, sv, tp, kp, wt_t, wk_t, wv_t, b2)


def _counts_dst(idx):
  d = idx.astype(jnp.int32).reshape(32, NPASS, PASS_ROWS, L)
  d = d + jnp.arange(PASS_ROWS, dtype=jnp.int32)[None, None, :, None] * VPAD
  d = d.transpose(0, 1, 3, 2).reshape(32, NPASS, PASS_POS)
  pad = jnp.zeros((32, NPASS, PASS_PAD - PASS_POS), jnp.int32)
  return jnp.concatenate([d, pad], axis=-1).reshape(32, NPASS, 13, 8, 16)


def kernel(sol_type_idx, sol_key_idx, sol_val_idx, type_table, key_table,
           val_table, W, b):
  idx_v = sol_val_idx.astype(jnp.int32).reshape(32, NCHUNK, CHUNK)
  dstc = jnp.stack([_counts_dst(sol_type_idx), _counts_dst(sol_key_idx)])
  pos = jnp.arange(ROWS_PER_W * L, dtype=jnp.int32) // L
  dmap = (jnp.arange(16, dtype=jnp.int32)[:, None] * ROWS_PER_W +
          pos[None, :]).reshape(16, NCHUNK, CHUNK)
  zeros2d = jnp.zeros((ROWS_PER_W, EMB_DIM), jnp.float32)
  zeros1d = jnp.zeros((PASS_ROWS * VPAD,), jnp.float32)

  ct, ck, sv = _sc_pool(idx_v, dstc, val_table, dmap, zeros2d, zeros1d)

  tp = jnp.zeros((VPAD, EMB_DIM), jnp.float32).at[:1000].set(type_table)
  kp = jnp.zeros((VPAD, EMB_DIM), jnp.float32).at[:1000].set(key_table)
  wT = W.T
  return _tc_head(ct.reshape(B, VPAD), ck.reshape(B, VPAD), sv, tp, kp,
                  wT[:EMB_DIM], wT[EMB_DIM:2 * EMB_DIM], wT[2 * EMB_DIM:],
                  b.reshape(1, HIDDEN))


# speedup vs baseline: 17.7540x; 1.6150x over previous
import functools

import jax
import jax.numpy as jnp
from jax import lax
from jax.experimental import pallas as pl
from jax.experimental.pallas import tpu as pltpu
from jax.experimental.pallas import tpu_sc as plsc

B, L = 4096, 50
EMB_DIM, HIDDEN = 128, 512
VPAD = 1024
CHUNK = 128
NCHUNK = (B // 32) * L // CHUNK
ROWS_PER_W = B // 32
NBUF = 2
PASS_ROWS = 32
NPASS = ROWS_PER_W // PASS_ROWS
PASS_POS = PASS_ROWS * L
PASS_PAD = 1664


def _sc_pool(idx_v, dstc, val_table, dmap, zeros2d, zeros1d):
  mesh = plsc.VectorSubcoreMesh(core_axis_name="c", subcore_axis_name="s")
  out_t = [
      jax.ShapeDtypeStruct((B, VPAD), jnp.float32),
      jax.ShapeDtypeStruct((B, VPAD), jnp.float32),
      jax.ShapeDtypeStruct((B, EMB_DIM), jnp.float32),
  ]

  @functools.partial(
      pl.kernel,
      mesh=mesh,
      out_type=out_t,
      compiler_params=pltpu.CompilerParams(needs_layout_passes=False),
      scratch_types=[
          pltpu.VMEM((NCHUNK, CHUNK), jnp.int32),
          pltpu.VMEM((NCHUNK, CHUNK), jnp.int32),
          pltpu.VMEM((PASS_PAD // 128, 128), jnp.int32),
          pltpu.VMEM((PASS_ROWS, VPAD), jnp.float32),
          pltpu.VMEM((NBUF * CHUNK, EMB_DIM), jnp.float32),
          pltpu.VMEM_SHARED((16 * ROWS_PER_W, EMB_DIM), jnp.float32),
      ] + [pltpu.SemaphoreType.DMA] * (2 * NBUF + 2),
  )
  def k(idx_v_hbm, dstc_hbm, vtab, dmap_hbm, zeros2d_hbm, zerosc_hbm,
        o_ct, o_ck, o_sv, idx_vm, dmap_vm, dst_vm, cnt_vm, rows_vm, acc_sh,
        *sems):
    gsems = sems[:NBUF]
    ssems = sems[NBUF:2 * NBUF]
    zsem, rsem = sems[2 * NBUF], sems[2 * NBUF + 1]
    s = lax.axis_index("s")
    wid = lax.axis_index("c") * 16 + s
    ones16 = jnp.ones((16,), jnp.float32)
    bufs = [rows_vm.at[pl.ds(j * CHUNK, CHUNK)] for j in range(NBUF)]

    rows16 = [lax.iota(jnp.int32, 16) + 16 * par for par in range(2)]

    pltpu.sync_copy(dmap_hbm.at[s], dmap_vm)
    pltpu.sync_copy(idx_v_hbm.at[wid], idx_vm)
    pltpu.sync_copy(zeros2d_hbm, acc_sh.at[pl.ds(s * ROWS_PER_W, ROWS_PER_W)])

    gsem, ssem = gsems[0], ssems[0]
    pltpu.async_copy(vtab.at[idx_vm.at[0]], bufs[0], gsem)

    def body(c, carry):
      off = lax.rem(c, 2) * CHUNK
      buf = rows_vm.at[pl.ds(off, CHUNK)]
      pltpu.make_async_copy(vtab.at[idx_vm.at[c]], buf, gsem).wait()

      @pl.when(c >= 1)
      def _():
        pltpu.make_async_copy(buf, acc_sh.at[dmap_vm.at[c]], ssem).wait()

      @pl.when(c <= NCHUNK - 2)
      def _():
        nbuf = rows_vm.at[pl.ds((CHUNK - off), CHUNK)]
        pltpu.async_copy(vtab.at[idx_vm.at[c + 1]], nbuf, gsem)

      pltpu.async_copy(buf, acc_sh.at[dmap_vm.at[c]], ssem, add=True)

      cm = lax.rem(c, 5)

      @pl.when(jnp.logical_and(cm == 3, c < 40))
      def _():
        @pl.when(c >= 8)
        def _():
          pltpu.make_async_copy(cnt_vm, o_ct.at[pl.ds(0, PASS_ROWS)],
                                rsem).wait()
        pltpu.async_copy(zerosc_hbm, cnt_vm, zsem)

      @pl.when(jnp.logical_and(cm == 0, jnp.logical_and(c >= 5, c <= 40)))
      def _():
        p = c // 5 - 1
        t = p // 4
        q = lax.rem(p, 4)
        pltpu.make_async_copy(zerosc_hbm, cnt_vm, zsem).wait()
        pltpu.sync_copy(dstc_hbm.at[t].at[wid].at[q], dst_vm)

        def cbody(r, carry):
          for j in range(8):
            col16 = dst_vm[r, pl.ds(j * 16, 16)]
            plsc.addupdate_scatter(cnt_vm, [rows16[j % 2], col16], ones16)
          return carry

        lax.fori_loop(0, 12, cbody, 0)
        for j in range(4):
          col16 = dst_vm[12, pl.ds(j * 16, 16)]
          plsc.addupdate_scatter(cnt_vm, [rows16[j % 2], col16], ones16)

        row0 = wid * ROWS_PER_W + q * PASS_ROWS

        @pl.when(t == 0)
        def _():
          pltpu.async_copy(cnt_vm, o_ct.at[pl.ds(row0, PASS_ROWS)], rsem)

        @pl.when(t == 1)
        def _():
          pltpu.async_copy(cnt_vm, o_ck.at[pl.ds(row0, PASS_ROWS)], rsem)

      return carry

    lax.fori_loop(0, NCHUNK, body, 0)
    pltpu.make_async_copy(bufs[0], acc_sh.at[dmap_vm.at[0]], ssem).wait()
    pltpu.make_async_copy(cnt_vm, o_ct.at[pl.ds(0, PASS_ROWS)], rsem).wait()
    pltpu.sync_copy(acc_sh.at[pl.ds(s * ROWS_PER_W, ROWS_PER_W)],
                    o_sv.at[pl.ds(wid * ROWS_PER_W, ROWS_PER_W)])

  return k(idx_v, dstc, val_table, dmap, zeros2d, zeros1d)


def _tc_head(ct, ck, sv, tp, kp, wt_t, wk_t, wv_t, b2):
  BM = 512
  grid = (B // BM,)

  def body(ct_r, ck_r, sv_r, tp_r, kp_r, wt_r, wk_r, wv_r, b_r, o_r):
    st = jnp.dot(ct_r[...], tp_r[...], preferred_element_type=jnp.float32)
    sk = jnp.dot(ck_r[...], kp_r[...], preferred_element_type=jnp.float32)
    acc = jnp.dot(st, wt_r[...], preferred_element_type=jnp.float32)
    acc += jnp.dot(sk, wk_r[...], preferred_element_type=jnp.float32)
    acc += jnp.dot(sv_r[...], wv_r[...], preferred_element_type=jnp.float32)
    o_r[...] = jnp.maximum(acc * (1.0 / L) + b_r[...], 0.0)

  return pl.pallas_call(
      body,
      grid=grid,
      in_specs=[
          pl.BlockSpec((BM, VPAD), lambda i: (i, 0)),
          pl.BlockSpec((BM, VPAD), lambda i: (i, 0)),
          pl.BlockSpec((BM, EMB_DIM), lambda i: (i, 0)),
          pl.BlockSpec((VPAD, EMB_DIM), lambda i: (0, 0)),
          pl.BlockSpec((VPAD, EMB_DIM), lambda i: (0, 0)),
          pl.BlockSpec((EMB_DIM, HIDDEN), lambda i: (0, 0)),
          pl.BlockSpec((EMB_DIM, HIDDEN), lambda i: (0, 0)),
          pl.BlockSpec((EMB_DIM, HIDDEN), lambda i: (0, 0)),
          pl.BlockSpec((1, HIDDEN), lambda i: (0, 0)),
      ],
      out_specs=pl.BlockSpec((BM, HIDDEN), lambda i: (i, 0)),
      out_shape=jax.ShapeDtypeStruct((B, HIDDEN), jnp.float32),
  )(ct, ck, sv, tp, kp, wt_t, wk_t, wv_t, b2)


def _counts_dst(idx):
  d = idx.astype(jnp.int32).reshape(32, NPASS, PASS_ROWS, L)
  d = d.transpose(0, 1, 3, 2).reshape(32, NPASS, PASS_POS)
  pad = jnp.zeros((32, NPASS, PASS_PAD - PASS_POS), jnp.int32)
  return jnp.concatenate([d, pad], axis=-1).reshape(32, NPASS, 13, 128)


def kernel(sol_type_idx, sol_key_idx, sol_val_idx, type_table, key_table,
           val_table, W, b):
  idx_v = sol_val_idx.astype(jnp.int32).reshape(32, NCHUNK, CHUNK)
  dstc = jnp.stack([_counts_dst(sol_type_idx), _counts_dst(sol_key_idx)])
  pos = jnp.arange(ROWS_PER_W * L, dtype=jnp.int32) // L
  dmap = (jnp.arange(16, dtype=jnp.int32)[:, None] * ROWS_PER_W +
          pos[None, :]).reshape(16, NCHUNK, CHUNK)
  zeros2d = jnp.zeros((ROWS_PER_W, EMB_DIM), jnp.float32)
  zerosc = jnp.zeros((PASS_ROWS, VPAD), jnp.float32)

  ct, ck, sv = _sc_pool(idx_v, dstc, val_table, dmap, zeros2d, zerosc)

  tp = jnp.zeros((VPAD, EMB_DIM), jnp.float32).at[:1000].set(type_table)
  kp = jnp.zeros((VPAD, EMB_DIM), jnp.float32).at[:1000].set(key_table)
  wT = W.T
  return _tc_head(ct, ck, sv, tp, kp,
                  wT[:EMB_DIM], wT[EMB_DIM:2 * EMB_DIM], wT[2 * EMB_DIM:],
                  b.reshape(1, HIDDEN))
